# baseline (speedup 1.0000x reference)
import functools

import jax
import jax.numpy as jnp
from jax import lax
from jax.experimental import pallas as pl
from jax.experimental.pallas import tpu as pltpu
from jax.experimental.pallas import tpu_sc as plsc

N, E, F, D, C = 10000, 320000, 128, 48, 40
NC, NS = 2, 16
NW = NC * NS
EW = E // NW
NPAD = 10240
NPT = NPAD // NS


def _segment_sum_sc(p, src, dst, zeros, width, k, depth, stage_table=False):
    nchunk = EW // k
    assert nchunk * k == EW and nchunk % depth == 0
    mesh = plsc.VectorSubcoreMesh(core_axis_name="c", subcore_axis_name="s")

    @functools.partial(
        pl.kernel,
        out_type=jax.ShapeDtypeStruct((NC, NPAD, width), jnp.float32),
        mesh=mesh,
        scratch_types=(
            [pltpu.VMEM((nchunk, k), jnp.int32),
             pltpu.VMEM((nchunk, k), jnp.int32)]
            + [pltpu.VMEM((k, width), jnp.float32)] * depth
            + [pltpu.VMEM_SHARED((NPAD, width), jnp.float32)]
            + ([pltpu.VMEM_SHARED((N, width), jnp.float32)] if stage_table
               else [])
            + [pltpu.SemaphoreType.DMA] * depth
        ),
        compiler_params=pltpu.CompilerParams(use_tc_tiling_on_sc=False),
    )
    def seg(p_hbm, src_hbm, dst_hbm, z_hbm, out_hbm, sidx, didx, *rest):
        rows = rest[:depth]
        acc = rest[depth]
        if stage_table:
            table = rest[depth + 1]
            sems = rest[depth + 2:]
        else:
            table = p_hbm
            sems = rest[depth + 1:]
        c = lax.axis_index("c")
        s = lax.axis_index("s")
        wid = c * NS + s
        pltpu.sync_copy(z_hbm.at[pl.ds(s * NPT, NPT)], acc.at[pl.ds(s * NPT, NPT)])
        pltpu.sync_copy(src_hbm.at[wid], sidx)
        pltpu.sync_copy(dst_hbm.at[wid], didx)
        if stage_table:
            @pl.when(s == 0)
            def _():
                pltpu.sync_copy(p_hbm, table)
        plsc.subcore_barrier()

        for b in range(depth):
            pltpu.async_copy(table.at[sidx.at[b]], rows[b], sems[b])

        def grp(g, carry):
            base = g * depth
            for b in range(depth):
                i = base + b
                pltpu.make_async_copy(p_hbm.at[sidx.at[0]], rows[b],
                                      sems[b]).wait()
                pltpu.sync_copy(rows[b], acc.at[didx.at[i]], add=True)

                @pl.when(i + depth < nchunk)
                def _():
                    pltpu.async_copy(table.at[sidx.at[i + depth]],
                                     rows[b], sems[b])
            return carry

        lax.fori_loop(0, nchunk // depth, grp, 0)
        plsc.subcore_barrier()
        pltpu.sync_copy(acc.at[pl.ds(s * NPT, NPT)],
                        out_hbm.at[c, pl.ds(s * NPT, NPT)])

    return seg(p, src, dst, zeros)


def _tc_relu(x):
    def body(x_ref, o_ref):
        o_ref[...] = jnp.maximum(x_ref[...], 0.0)

    return pl.pallas_call(
        body, out_shape=jax.ShapeDtypeStruct(x.shape, jnp.float32))(x)


def _gine_bn_block(h_ref, s_ref, wa_ref, ba_ref, wb_ref, bb_ref, g_ref, be_ref):
    agg = s_ref[0, :N, :] + s_ref[1, :N, :]
    t = jnp.maximum(
        lax.dot(h_ref[...] + agg, wa_ref[...],
                preferred_element_type=jnp.float32) + ba_ref[...], 0.0)
    u = lax.dot(t, wb_ref[...], preferred_element_type=jnp.float32) + bb_ref[...]
    u = jnp.maximum(u, 0.0)
    m = jnp.mean(u, axis=0, keepdims=True)
    v = jnp.mean((u - m) ** 2, axis=0, keepdims=True)
    return (u - m) / jnp.sqrt(v + 1e-5) * g_ref[...] + be_ref[...]


def _tc_mid(x, s1, w1a, b1a, w1b, b1b, g1, be1):
    def body(x_ref, s_ref, wa_ref, ba_ref, wb_ref, bb_ref, g_ref, be_ref,
             h_ref, r_ref):
        h = _gine_bn_block(x_ref, s_ref, wa_ref, ba_ref, wb_ref, bb_ref,
                           g_ref, be_ref)
        h_ref[...] = h
        r_ref[...] = jnp.maximum(h, 0.0)

    return pl.pallas_call(
        body,
        out_shape=(jax.ShapeDtypeStruct((N, D), jnp.float32),
                   jax.ShapeDtypeStruct((N, D), jnp.float32)),
    )(x, s1, w1a, b1a, w1b, b1b, g1, be1)


def _tc_fin(h1, s2, w2a, b2a, w2b, b2b, g2, be2, wf1, bf1, wf2, bf2):
    def body(h1_ref, s_ref, wa_ref, ba_ref, wb_ref, bb_ref, g_ref, be_ref,
             wf1_ref, bf1_ref, wf2_ref, bf2_ref, out_ref):
        h = _gine_bn_block(h1_ref, s_ref, wa_ref, ba_ref, wb_ref, bb_ref,
                           g_ref, be_ref)
        f = jnp.maximum(
            lax.dot(h, wf1_ref[...], preferred_element_type=jnp.float32)
            + bf1_ref[...], 0.0)
        out_ref[...] = (lax.dot(f, wf2_ref[...], preferred_element_type=jnp.float32)
                        + bf2_ref[...])

    return pl.pallas_call(
        body,
        out_shape=jax.ShapeDtypeStruct((N, C), jnp.float32),
    )(h1, s2, w2a, b2a, w2b, b2b, g2, be2, wf1, bf1, wf2, bf2)


def kernel(x, edge_index, W1a, b1a, W1b, b1b, g1, be1, W2a, b2a, W2b, b2b,
           g2, be2, Wf1, bf1, Wf2, bf2):
    K1, DEPTH1 = 50, 4
    K2, DEPTH2 = 125, 8
    src = edge_index[0].astype(jnp.int32)
    dst = edge_index[1].astype(jnp.int32)
    src1 = src.reshape(NW, EW // K1, K1)
    dst1 = dst.reshape(NW, EW // K1, K1)
    src2 = src.reshape(NW, EW // K2, K2)
    dst2 = dst.reshape(NW, EW // K2, K2)
    zeros_f = jnp.zeros((NPAD, F), jnp.float32)
    zeros_d = jnp.zeros((NPAD, D), jnp.float32)

    r0 = _tc_relu(x)
    s1 = _segment_sum_sc(r0, src1, dst1, zeros_f, F, K1, DEPTH1)
    h1, r1 = _tc_mid(x, s1, W1a, b1a.reshape(1, D), W1b, b1b.reshape(1, D),
                     g1.reshape(1, D), be1.reshape(1, D))
    s2 = _segment_sum_sc(r1, src2, dst2, zeros_d, D, K2, DEPTH2,
                         stage_table=True)
    return _tc_fin(h1, s2, W2a, b2a.reshape(1, D), W2b, b2b.reshape(1, D),
                   g2.reshape(1, D), be2.reshape(1, D),
                   Wf1, bf1.reshape(1, D), Wf2, bf2.reshape(1, C))


# speedup vs baseline: 1.0777x; 1.0777x over previous
import functools

import jax
import jax.numpy as jnp
from jax import lax
from jax.experimental import pallas as pl
from jax.experimental.pallas import tpu as pltpu
from jax.experimental.pallas import tpu_sc as plsc

N, E, F, D, C = 10000, 320000, 128, 48, 40
NC, NS = 2, 16
NW = NC * NS
EW = E // NW
NPAD = 10240
NPT = NPAD // NS


def _segment_sum_sc(p, src, dst, zeros, width, k, depth):
    nchunk = EW // k
    assert nchunk * k == EW and nchunk % depth == 0
    mesh = plsc.VectorSubcoreMesh(core_axis_name="c", subcore_axis_name="s")

    @functools.partial(
        pl.kernel,
        out_type=jax.ShapeDtypeStruct((NC, NPAD, width), jnp.float32),
        mesh=mesh,
        scratch_types=(
            [pltpu.VMEM((nchunk, k), jnp.int32),
             pltpu.VMEM((nchunk, k), jnp.int32)]
            + [pltpu.VMEM((k, width), jnp.float32)] * depth
            + [pltpu.VMEM_SHARED((NPAD, width), jnp.float32)]
            + [pltpu.SemaphoreType.DMA] * depth
        ),
        compiler_params=pltpu.CompilerParams(use_tc_tiling_on_sc=False),
    )
    def seg(p_hbm, src_hbm, dst_hbm, z_hbm, out_hbm, sidx, didx, *rest):
        rows = rest[:depth]
        acc = rest[depth]
        table = p_hbm
        sems = rest[depth + 1:]
        c = lax.axis_index("c")
        s = lax.axis_index("s")
        wid = c * NS + s
        pltpu.sync_copy(z_hbm.at[pl.ds(s * NPT, NPT)], acc.at[pl.ds(s * NPT, NPT)])
        pltpu.sync_copy(src_hbm.at[wid], sidx)
        pltpu.sync_copy(dst_hbm.at[wid], didx)
        plsc.subcore_barrier()

        for b in range(depth):
            pltpu.async_copy(table.at[sidx.at[b]], rows[b], sems[b])

        def grp(g, carry):
            base = g * depth
            for b in range(depth):
                i = base + b
                pltpu.make_async_copy(p_hbm.at[sidx.at[0]], rows[b],
                                      sems[b]).wait()
                pltpu.sync_copy(rows[b], acc.at[didx.at[i]], add=True)

                @pl.when(i + depth < nchunk)
                def _():
                    pltpu.async_copy(table.at[sidx.at[i + depth]],
                                     rows[b], sems[b])
            return carry

        lax.fori_loop(0, nchunk // depth, grp, 0)
        plsc.subcore_barrier()
        pltpu.sync_copy(acc.at[pl.ds(s * NPT, NPT)],
                        out_hbm.at[c, pl.ds(s * NPT, NPT)])

    return seg(p, src, dst, zeros)


def _tc_relu(x):
    def body(x_ref, o_ref):
        o_ref[...] = jnp.maximum(x_ref[...], 0.0)

    return pl.pallas_call(
        body, out_shape=jax.ShapeDtypeStruct(x.shape, jnp.float32))(x)


def _gine_bn_block(h_ref, s_ref, wa_ref, ba_ref, wb_ref, bb_ref, g_ref, be_ref):
    agg = s_ref[0, :N, :] + s_ref[1, :N, :]
    t = jnp.maximum(
        lax.dot(h_ref[...] + agg, wa_ref[...],
                preferred_element_type=jnp.float32) + ba_ref[...], 0.0)
    u = lax.dot(t, wb_ref[...], preferred_element_type=jnp.float32) + bb_ref[...]
    u = jnp.maximum(u, 0.0)
    m = jnp.mean(u, axis=0, keepdims=True)
    v = jnp.mean((u - m) ** 2, axis=0, keepdims=True)
    return (u - m) / jnp.sqrt(v + 1e-5) * g_ref[...] + be_ref[...]


def _tc_mid(x, s1, w1a, b1a, w1b, b1b, g1, be1):
    def body(x_ref, s_ref, wa_ref, ba_ref, wb_ref, bb_ref, g_ref, be_ref,
             h_ref, r_ref):
        h = _gine_bn_block(x_ref, s_ref, wa_ref, ba_ref, wb_ref, bb_ref,
                           g_ref, be_ref)
        h_ref[...] = h
        r_ref[...] = jnp.maximum(h, 0.0)

    return pl.pallas_call(
        body,
        out_shape=(jax.ShapeDtypeStruct((N, D), jnp.float32),
                   jax.ShapeDtypeStruct((N, D), jnp.float32)),
    )(x, s1, w1a, b1a, w1b, b1b, g1, be1)


def _tc_fin(h1, s2, w2a, b2a, w2b, b2b, g2, be2, wf1, bf1, wf2, bf2):
    def body(h1_ref, s_ref, wa_ref, ba_ref, wb_ref, bb_ref, g_ref, be_ref,
             wf1_ref, bf1_ref, wf2_ref, bf2_ref, out_ref):
        h = _gine_bn_block(h1_ref, s_ref, wa_ref, ba_ref, wb_ref, bb_ref,
                           g_ref, be_ref)
        f = jnp.maximum(
            lax.dot(h, wf1_ref[...], preferred_element_type=jnp.float32)
            + bf1_ref[...], 0.0)
        out_ref[...] = (lax.dot(f, wf2_ref[...], preferred_element_type=jnp.float32)
                        + bf2_ref[...])

    return pl.pallas_call(
        body,
        out_shape=jax.ShapeDtypeStruct((N, C), jnp.float32),
    )(h1, s2, w2a, b2a, w2b, b2b, g2, be2, wf1, bf1, wf2, bf2)


def kernel(x, edge_index, W1a, b1a, W1b, b1b, g1, be1, W2a, b2a, W2b, b2b,
           g2, be2, Wf1, bf1, Wf2, bf2):
    K1, DEPTH1 = 50, 4
    K2, DEPTH2 = 125, 4
    src = edge_index[0].astype(jnp.int32)
    dst = edge_index[1].astype(jnp.int32)
    src1 = src.reshape(NW, EW // K1, K1)
    dst1 = dst.reshape(NW, EW // K1, K1)
    src2 = src.reshape(NW, EW // K2, K2)
    dst2 = dst.reshape(NW, EW // K2, K2)
    zeros_f = jnp.zeros((NPAD, F), jnp.float32)
    zeros_d = jnp.zeros((NPAD, D), jnp.float32)

    r0 = _tc_relu(x)
    s1 = _segment_sum_sc(r0, src1, dst1, zeros_f, F, K1, DEPTH1)
    h1, r1 = _tc_mid(x, s1, W1a, b1a.reshape(1, D), W1b, b1b.reshape(1, D),
                     g1.reshape(1, D), be1.reshape(1, D))
    s2 = _segment_sum_sc(r1, src2, dst2, zeros_d, D, K2, DEPTH2)
    return _tc_fin(h1, s2, W2a, b2a.reshape(1, D), W2b, b2b.reshape(1, D),
                   g2.reshape(1, D), be2.reshape(1, D),
                   Wf1, bf1.reshape(1, D), Wf2, bf2.reshape(1, C))
